# Initial kernel scaffold; baseline (speedup 1.0000x reference)
import jax
import jax.numpy as jnp
from jax.experimental import pallas as pl


def kernel(first, mid, last, inv_i, seq_len, emb_table, W, b):
    raise NotImplementedError("write your pallas kernel here")


# speedup vs baseline: 2.8288x; 2.8288x over previous
import functools

import jax
import jax.numpy as jnp
from jax import lax
from jax.experimental import pallas as pl
from jax.experimental.pallas import tpu as pltpu
from jax.experimental.pallas import tpu_sc as plsc

SEQ = 1024
NC, NS = 2, 16
NW = NC * NS
TOK_PER_TILE_B = 128
TOK_PER_CHUNK = 8
K = 14
CH = TOK_PER_CHUNK * K
NCH = TOK_PER_TILE_B // TOK_PER_CHUNK


def _matmul_block(emb_ref, w_ref, b_ref, out_ref):
    acc = jnp.dot(emb_ref[...], w_ref[...], preferred_element_type=jnp.float32)
    sel = (pl.program_id(0) == 0).astype(jnp.float32)
    out_ref[...] = acc + sel * b_ref[...]


def _stage_a(emb_p, W, b2):
    VP, H = emb_p.shape
    O = W.shape[1]
    nrb = VP // 512
    return pl.pallas_call(
        _matmul_block,
        grid=(3, nrb),
        in_specs=[
            pl.BlockSpec((512, H), lambda k, i: (i, 0)),
            pl.BlockSpec((512, O), lambda k, i: (k, 0)),
            pl.BlockSpec((1, O), lambda k, i: (0, 0)),
        ],
        out_specs=pl.BlockSpec((512, O), lambda k, i: (k * nrb + i, 0)),
        out_shape=jax.ShapeDtypeStruct((3 * VP, O), jnp.float32),
    )(emb_p, W, b2)


def _stage_b(tcat, idx3, U, O):
    NG = O // 16

    @functools.partial(
        pl.kernel,
        mesh=plsc.VectorSubcoreMesh(core_axis_name="c", subcore_axis_name="s"),
        out_type=jax.ShapeDtypeStruct((U, O), jnp.float32),
        scratch_types=[
            pltpu.VMEM((NCH, CH), jnp.int32),
            pltpu.VMEM((2, CH, O), jnp.float32),
            pltpu.VMEM((2, TOK_PER_CHUNK, O), jnp.float32),
            pltpu.SemaphoreType.DMA,
            pltpu.SemaphoreType.DMA,
            pltpu.SemaphoreType.DMA,
        ],
    )
    def body(tcat_hbm, idx_hbm, out_hbm, idx_v, stage_v, outst_v, gsem, ws0, ws1):
        cid = lax.axis_index("c")
        sid = lax.axis_index("s")
        wid = cid * NS + sid
        row0 = wid * TOK_PER_TILE_B
        pltpu.sync_copy(idx_hbm.at[wid], idx_v)

        wsems = (ws0, ws1)
        gathers = [None, None]
        writes = [None, None]
        gathers[0] = pltpu.async_copy(
            tcat_hbm.at[idx_v.at[0]], stage_v.at[0], gsem
        )
        for j in range(NCH):
            p = j % 2
            gathers[p].wait()
            if j + 1 < NCH:
                gathers[1 - p] = pltpu.async_copy(
                    tcat_hbm.at[idx_v.at[j + 1]], stage_v.at[1 - p], gsem
                )
            if writes[p] is not None:
                writes[p].wait()

            def red(g, _):
                c0 = g * 16
                for t in range(TOK_PER_CHUNK):
                    v = stage_v[p, t * K, pl.ds(c0, 16)]
                    for k in range(1, K):
                        v = v + stage_v[p, t * K + k, pl.ds(c0, 16)]
                    outst_v[p, t, pl.ds(c0, 16)] = v
                return 0

            lax.fori_loop(0, NG, red, 0)
            writes[p] = pltpu.async_copy(
                outst_v.at[p],
                out_hbm.at[pl.ds(row0 + j * TOK_PER_CHUNK, TOK_PER_CHUNK)],
                wsems[p],
            )
        for w in writes:
            if w is not None:
                w.wait()

    return body(tcat, idx3)


def _stage_c(out_u, inv3, T, O):
    tok_per_tile = T // NW
    nchunk = tok_per_tile // 128
    bsz = T // SEQ
    tiles_per_seq = SEQ // tok_per_tile

    @functools.partial(
        pl.kernel,
        mesh=plsc.VectorSubcoreMesh(core_axis_name="c", subcore_axis_name="s"),
        compiler_params=pltpu.CompilerParams(use_tc_tiling_on_sc=False),
        out_type=jax.ShapeDtypeStruct((bsz * (SEQ + 2), O), jnp.float32),
        scratch_types=[
            pltpu.VMEM((nchunk, 128), jnp.int32),
            pltpu.VMEM((128, O), jnp.float32),
            pltpu.SemaphoreType.DMA,
        ],
    )
    def body(src_hbm, inv_hbm, out_hbm, idx_v, stage_v, sem):
        cid = lax.axis_index("c")
        sid = lax.axis_index("s")
        wid = cid * NS + sid
        seq = wid // tiles_per_seq
        lane = wid % tiles_per_seq
        dst_base = seq * (SEQ + 2) + 1 + lane * tok_per_tile
        pltpu.sync_copy(inv_hbm.at[wid], idx_v)
        for j in range(nchunk):
            pltpu.async_copy(src_hbm.at[idx_v.at[j]], stage_v, sem).wait()
            pltpu.sync_copy(stage_v, out_hbm.at[pl.ds(dst_base + j * 128, 128)])
        zv = jnp.zeros((16,), jnp.float32)

        def zr(i, _):
            stage_v[0, pl.ds(i * 16, 16)] = zv
            return 0

        lax.fori_loop(0, O // 16, zr, 0)

        @pl.when(lane == 0)
        def _():
            pltpu.sync_copy(stage_v.at[pl.ds(0, 1)],
                            out_hbm.at[pl.ds(seq * (SEQ + 2), 1)])

        @pl.when(lane == tiles_per_seq - 1)
        def _():
            pltpu.sync_copy(stage_v.at[pl.ds(0, 1)],
                            out_hbm.at[pl.ds(seq * (SEQ + 2) + SEQ + 1, 1)])

    return body(out_u, inv3)


def kernel(first, mid, last, inv_i, seq_len, emb_table, W, b):
    V, H = emb_table.shape
    O = W.shape[1]
    U = first.shape[0]
    T = inv_i.shape[0]
    bsz = T // SEQ
    VP = 4096

    emb_p = jnp.pad(emb_table, ((0, VP - V), (0, 0)))
    tcat = _stage_a(emb_p, W, b.reshape(1, O))

    first = first.astype(jnp.int32)
    mid = mid.astype(jnp.int32)
    last = last.astype(jnp.int32)
    inv_i = inv_i.astype(jnp.int32)

    idx_all = jnp.concatenate(
        [first[:, None], mid + VP, last[:, None] + 2 * VP], axis=1
    )
    idx3 = idx_all.reshape(NW, NCH, CH)
    out_u = _stage_b(tcat, idx3, U, O)

    inv3 = inv_i.reshape(NW, T // NW // 128, 128)
    flat = _stage_c(out_u, inv3, T, O)
    return flat.reshape(bsz, SEQ + 2, O)
